# baseline (speedup 1.0000x reference)
import functools

import jax
import jax.numpy as jnp
from jax import lax
from jax.experimental import pallas as pl
from jax.experimental.pallas import tpu as pltpu
from jax.experimental.pallas import tpu_sc as plsc

_B = 128
_V = 100000
_A_PAD = 112
_NC = 2
_NS = 16
_ROWS = _B // (_NC * _NS)
_L = 16


def _patch_body(out_hbm, scores_hbm, allowed_hbm, allowed_v, idx_v, vals_v,
                sem):
    wid = lax.axis_index("s") * _NC + lax.axis_index("c")
    pltpu.sync_copy(allowed_hbm, allowed_v)
    for r in range(_ROWS):
        base = (wid * _ROWS + r) * _V
        for c in range(_A_PAD // _L):
            a16 = allowed_v[pl.ds(_L * c, _L)]
            idx_v[pl.ds(r * _A_PAD + _L * c, _L)] = a16 + base
    pltpu.async_copy(scores_hbm.at[idx_v], vals_v, sem).wait()
    pltpu.async_copy(vals_v, out_hbm.at[idx_v], sem).wait()


_patch = functools.partial(
    pl.kernel,
    mesh=plsc.VectorSubcoreMesh(
        core_axis_name="c", subcore_axis_name="s",
        num_cores=_NC, num_subcores=_NS),
    scratch_types=[
        pltpu.VMEM((_A_PAD,), jnp.int32),
        pltpu.VMEM((_ROWS * _A_PAD,), jnp.int32),
        pltpu.VMEM((_ROWS * _A_PAD,), jnp.float32),
        pltpu.SemaphoreType.DMA,
    ],
)(_patch_body)


def kernel(input_ids, scores, allowed_token_ids):
    del input_ids
    a = allowed_token_ids.astype(jnp.int32)
    a_pad = jnp.concatenate(
        [a, jnp.broadcast_to(a[-1:], (_A_PAD - a.shape[0],))])
    canvas = jnp.full((_B * _V,), -jnp.inf, jnp.float32)
    out_ref = jax.new_ref(canvas)
    _patch(out_ref, scores.reshape(-1), a_pad)
    return out_ref[...].reshape(_B, _V)


# speedup vs baseline: 1.0173x; 1.0173x over previous
import functools

import jax
import jax.numpy as jnp
from jax import lax
from jax.experimental import pallas as pl
from jax.experimental.pallas import tpu as pltpu
from jax.experimental.pallas import tpu_sc as plsc

_B = 128
_V = 100000
_A_PAD = 112
_NC = 2
_NS = 16
_ROWS = _B // (_NC * _NS)
_L = 16


def _patch_body(out_hbm, scores_hbm, allowed_hbm, allowed_v, idx_v, vals_v,
                sem):
    wid = lax.axis_index("s") * _NC + lax.axis_index("c")
    pltpu.sync_copy(allowed_hbm, allowed_v)
    for r in range(_ROWS):
        base = (wid * _ROWS + r) * _V
        for c in range(_A_PAD // _L):
            a16 = allowed_v[pl.ds(_L * c, _L)]
            idx_v[pl.ds(r * _A_PAD + _L * c, _L)] = a16 + base
    pltpu.async_copy(scores_hbm.at[idx_v], vals_v, sem).wait()
    pltpu.async_copy(vals_v, out_hbm.at[idx_v], sem).wait()


_patch = functools.partial(
    pl.kernel,
    mesh=plsc.VectorSubcoreMesh(
        core_axis_name="c", subcore_axis_name="s",
        num_cores=_NC, num_subcores=_NS),
    scratch_types=[
        pltpu.VMEM((_A_PAD,), jnp.int32),
        pltpu.VMEM((_ROWS * _A_PAD,), jnp.int32),
        pltpu.VMEM((_ROWS * _A_PAD,), jnp.float32),
        pltpu.SemaphoreType.DMA,
    ],
)(_patch_body)


def kernel(input_ids, scores, allowed_token_ids):
    a = allowed_token_ids.astype(jnp.int32)
    a_pad = jnp.concatenate(
        [a, jnp.broadcast_to(a[-1:], (_A_PAD - a.shape[0],))])
    zero = (input_ids[0, 0] * 0).astype(jnp.float32)
    canvas = jnp.full((_B * _V,), -jnp.inf, jnp.float32) + zero
    out_ref = jax.new_ref(canvas)
    _patch(out_ref, scores.reshape(-1), a_pad)
    return jax.freeze(out_ref).reshape(_B, _V)
